# baseline (speedup 1.0000x reference)
import functools
import math

import jax
import jax.numpy as jnp
from jax import lax
from jax.experimental import pallas as pl
from jax.experimental.pallas import tpu as pltpu
from jax.experimental.pallas import tpu_sc as plsc

_N = 10000
_NPAD = 10240
_E = 640000
_FIN = 128
_HID = 64
_CLS = 7
_K = 4

_NC, _NS, _LANES = 2, 16, 16
_NW = _NC * _NS
_B = 128
_NCHUNK = 158
_EPT = _B * _NCHUNK
_EREAL = _E // _NW
_RPT = _NPAD // _NS

_COS = [math.cos(k * math.pi / _K) for k in range(_K)]
_SIN = [math.sin(k * math.pi / _K) for k in range(_K)]


def _mesh():
    return plsc.VectorSubcoreMesh(
        core_axis_name="c", subcore_axis_name="s",
        num_cores=_NC, num_subcores=_NS)


def _deg_body(dst_hbm, w_hbm, out_hbm, dstb, wb, zb, acc, lsem0, lsem1):
    c = lax.axis_index("c")
    s = lax.axis_index("s")
    wid = c * _NS + s
    lsem = (lsem0, lsem1)

    def zloop(i, carry):
        zb[pl.ds(i * 16, 16)] = jnp.zeros((16,), jnp.float32)
        return carry
    lax.fori_loop(0, _RPT // 16, zloop, 0)
    pltpu.sync_copy(zb, acc.at[pl.ds(s * _RPT, _RPT)])
    plsc.subcore_barrier()

    base = wid * _EPT

    def start_loads(i, b):
        off = base + i * _B
        pltpu.async_copy(dst_hbm.at[pl.ds(off, _B)], dstb.at[b], lsem[b])
        pltpu.async_copy(w_hbm.at[pl.ds(off, _B)], wb.at[b], lsem[b])

    def wait_loads(i, b):
        off = base + i * _B
        pltpu.make_async_copy(
            dst_hbm.at[pl.ds(off, _B)], dstb.at[b], lsem[b]).wait()
        pltpu.make_async_copy(
            w_hbm.at[pl.ds(off, _B)], wb.at[b], lsem[b]).wait()

    start_loads(0, 0)
    start_loads(1, 1)

    def macro(m, carry):
        for b in (0, 1):
            i = m * 2 + b
            wait_loads(i, b)
            pltpu.sync_copy(wb.at[b], acc.at[dstb.at[b]], add=True)

            @pl.when(i < _NCHUNK - 2)
            def _():
                start_loads(i + 2, b)
        return carry
    lax.fori_loop(0, _NCHUNK // 2, macro, 0)
    plsc.subcore_barrier()
    pltpu.sync_copy(acc.at[pl.ds(s * _RPT, _RPT)],
                    out_hbm.at[c, pl.ds(s * _RPT, _RPT)])


_deg_call = functools.partial(
    pl.kernel,
    out_type=jax.ShapeDtypeStruct((_NC, _NPAD), jnp.float32),
    mesh=_mesh(),
    scratch_types=[
        pltpu.VMEM((2, _B), jnp.int32),
        pltpu.VMEM((2, _B), jnp.float32),
        pltpu.VMEM((_RPT,), jnp.float32),
        pltpu.VMEM_SHARED((_NPAD,), jnp.float32),
        pltpu.SemaphoreType.DMA,
        pltpu.SemaphoreType.DMA,
    ],
)(_deg_body)


_NCHT = _NW * _NCHUNK


def _norm_body(src_hbm, dst_hbm, w_hbm, dinv_hbm, packi_hbm, packn_hbm,
               srcb, dstb, wb, gsb, gdb, pkb, nb,
               lsem0, lsem1, gsem0, gsem1, osem0, osem1):
    c = lax.axis_index("c")
    s = lax.axis_index("s")
    wid = c * _NS + s
    base = wid * _EPT
    lsem = (lsem0, lsem1)
    gsem = (gsem0, gsem1)
    osem = (osem0, osem1)

    def start_loads(i, b):
        off = base + i * _B
        pltpu.async_copy(src_hbm.at[pl.ds(off, _B)], srcb.at[b], lsem[b])
        pltpu.async_copy(dst_hbm.at[pl.ds(off, _B)], dstb.at[b], lsem[b])
        pltpu.async_copy(w_hbm.at[pl.ds(off, _B)], wb.at[b], lsem[b])

    def wait_loads(i, b):
        off = base + i * _B
        pltpu.make_async_copy(
            src_hbm.at[pl.ds(off, _B)], srcb.at[b], lsem[b]).wait()
        pltpu.make_async_copy(
            dst_hbm.at[pl.ds(off, _B)], dstb.at[b], lsem[b]).wait()
        pltpu.make_async_copy(
            w_hbm.at[pl.ds(off, _B)], wb.at[b], lsem[b]).wait()

    def start_gathers(b):
        pltpu.async_copy(dinv_hbm.at[srcb.at[b]], gsb.at[b], gsem[b])
        pltpu.async_copy(dinv_hbm.at[dstb.at[b]], gdb.at[b], gsem[b])

    def wait_gathers(b):
        pltpu.make_async_copy(
            dinv_hbm.at[srcb.at[b]], gsb.at[b], gsem[b]).wait()
        pltpu.make_async_copy(
            dinv_hbm.at[dstb.at[b]], gdb.at[b], gsem[b]).wait()

    def start_out(i, b):
        cid = wid * _NCHUNK + i
        pltpu.async_copy(pkb.at[b], packi_hbm.at[cid], osem[b])
        pltpu.async_copy(nb.at[b], packn_hbm.at[cid], osem[b])

    def wait_out(i, b):
        cid = wid * _NCHUNK + i
        pltpu.make_async_copy(
            pkb.at[b], packi_hbm.at[cid], osem[b]).wait()
        pltpu.make_async_copy(
            nb.at[b], packn_hbm.at[cid], osem[b]).wait()

    start_loads(0, 0)
    start_loads(1, 1)
    wait_loads(0, 0)
    start_gathers(0)

    def macro(m, carry):
        for b in (0, 1):
            i = m * 2 + b
            q = 1 - b

            @pl.when(i < _NCHUNK - 1)
            def _():
                wait_loads(i + 1, q)
                start_gathers(q)

            wait_gathers(b)

            @pl.when(i >= 2)
            def _():
                wait_out(i - 2, b)

            def vec(k, carry2):
                sl = pl.ds(k * 16, 16)
                pkb[b, 0, sl] = srcb[b, sl]
                pkb[b, 1, sl] = dstb[b, sl]
                nb[b, sl] = gsb[b, sl] * wb[b, sl] * gdb[b, sl]
                return carry2
            lax.fori_loop(0, _B // 16, vec, 0)
            start_out(i, b)

            @pl.when(i < _NCHUNK - 2)
            def _():
                start_loads(i + 2, b)
        return carry
    lax.fori_loop(0, _NCHUNK // 2, macro, 0)
    wait_out(_NCHUNK - 2, 0)
    wait_out(_NCHUNK - 1, 1)


_norm_call = functools.partial(
    pl.kernel,
    out_type=[jax.ShapeDtypeStruct((_NCHT, 2, _B), jnp.int32),
              jax.ShapeDtypeStruct((_NCHT, _B), jnp.float32)],
    mesh=_mesh(),
    scratch_types=[
        pltpu.VMEM((2, _B), jnp.int32),
        pltpu.VMEM((2, _B), jnp.int32),
        pltpu.VMEM((2, _B), jnp.float32),
        pltpu.VMEM((2, _B), jnp.float32),
        pltpu.VMEM((2, _B), jnp.float32),
        pltpu.VMEM((2, 2, _B), jnp.int32),
        pltpu.VMEM((2, _B), jnp.float32),
        pltpu.SemaphoreType.DMA,
        pltpu.SemaphoreType.DMA,
        pltpu.SemaphoreType.DMA,
        pltpu.SemaphoreType.DMA,
        pltpu.SemaphoreType.DMA,
        pltpu.SemaphoreType.DMA,
    ],
)(_norm_body)


def _make_prop(F):
    def body(packi_hbm, packn_hbm, h_hbm, out_hbm,
             pk, pn, rows, dstc, acc, *sems):
        c = lax.axis_index("c")
        s = lax.axis_index("s")
        wid = c * _NS + s
        psem = sems[0:2]
        gsem = sems[2:4]
        ssem = sems[4:6]

        def zr_loop(i, carry):
            for j in range(F // 16):
                rows[0, i, pl.ds(j * 16, 16)] = jnp.zeros((16,), jnp.float32)
            return carry
        lax.fori_loop(0, _B, zr_loop, 0)

        def zcp(kk, carry):
            pltpu.sync_copy(rows.at[0],
                            acc.at[pl.ds(s * _RPT + kk * _B, _B)])
            return carry
        lax.fori_loop(0, _RPT // _B, zcp, 0)
        plsc.subcore_barrier()

        cbase = wid * _NCHUNK

        def start_pack(i, b):
            pltpu.async_copy(packi_hbm.at[cbase + i], pk.at[b], psem[b])
            pltpu.async_copy(packn_hbm.at[cbase + i], pn.at[b], psem[b])

        def wait_pack(i, b):
            pltpu.make_async_copy(
                packi_hbm.at[cbase + i], pk.at[b], psem[b]).wait()
            pltpu.make_async_copy(
                packn_hbm.at[cbase + i], pn.at[b], psem[b]).wait()

        def start_gather(b):
            pltpu.async_copy(h_hbm.at[pk.at[b, 0]], rows.at[b], gsem[b])

        def wait_gather(b):
            pltpu.make_async_copy(
                h_hbm.at[pk.at[b, 0]], rows.at[b], gsem[b]).wait()

        def start_scatter(b):
            pltpu.async_copy(rows.at[b], acc.at[dstc.at[b]], ssem[b],
                             add=True)

        def wait_scatter(b):
            pltpu.make_async_copy(rows.at[b], acc.at[dstc.at[b]],
                                  ssem[b]).wait()

        start_pack(0, 0)
        start_pack(1, 1)
        wait_pack(0, 0)
        start_gather(0)

        def macro(m, carry):
            for b in (0, 1):
                i = m * 2 + b
                q = 1 - b

                @pl.when(i > 0)
                def _():
                    wait_scatter(q)

                @pl.when(i < _NCHUNK - 1)
                def _():
                    wait_pack(i + 1, q)
                    start_gather(q)

                wait_gather(b)

                def scale(g, carry2):
                    sl = pl.ds(g * 16, 16)
                    dstc[b, sl] = pk[b, 1, sl]
                    nv16 = pn[b, sl]
                    for r in range(16):
                        nv = nv16[r]
                        row = g * 16 + r
                        for j in range(F // 16):
                            slj = pl.ds(j * 16, 16)
                            rows[b, row, slj] = rows[b, row, slj] * nv
                    return carry2
                lax.fori_loop(0, _B // 16, scale, 0)

                start_scatter(b)

                @pl.when(i < _NCHUNK - 2)
                def _():
                    start_pack(i + 2, b)
            return carry
        lax.fori_loop(0, _NCHUNK // 2, macro, 0)
        wait_scatter((_NCHUNK - 1) % 2)
        plsc.subcore_barrier()
        pltpu.sync_copy(acc.at[pl.ds(s * _RPT, _RPT)],
                        out_hbm.at[c, pl.ds(s * _RPT, _RPT)])

    return functools.partial(
        pl.kernel,
        out_type=jax.ShapeDtypeStruct((_NC, _NPAD, F), jnp.float32),
        mesh=_mesh(),
        scratch_types=[
            pltpu.VMEM((2, 2, _B), jnp.int32),
            pltpu.VMEM((2, _B), jnp.float32),
            pltpu.VMEM((2, _B, F), jnp.float32),
            pltpu.VMEM((2, _B), jnp.int32),
            pltpu.VMEM_SHARED((_NPAD, F), jnp.float32),
        ] + [pltpu.SemaphoreType.DMA] * 6,
    )(body)


_prop128 = _make_prop(_FIN)


def _dinv_body(deg_ref, out_ref):
    d = deg_ref[0:1, :] + deg_ref[1:2, :]
    out_ref[:, :] = jnp.where(
        d > 0, lax.rsqrt(jnp.maximum(d, 1e-12)), 0.0)


_dinv_call = pl.pallas_call(
    _dinv_body,
    out_shape=jax.ShapeDtypeStruct((1, _NPAD), jnp.float32),
)


def _make_comb(F, with_base):
    BR = 2048
    grid = _NPAD // BR

    if with_base:
        def body(p_ref, base_ref, out_ref):
            ssum = p_ref[0, :, :] + p_ref[1, :, :]
            out_ref[:, :] = -2.0 * ssum - base_ref[:, :]
        in_specs = [
            pl.BlockSpec((2, BR, F), lambda i: (0, i, 0)),
            pl.BlockSpec((BR, F), lambda i: (i, 0)),
        ]
    else:
        def body(p_ref, out_ref):
            out_ref[:, :] = -(p_ref[0, :, :] + p_ref[1, :, :])
        in_specs = [pl.BlockSpec((2, BR, F), lambda i: (0, i, 0))]

    return pl.pallas_call(
        body,
        grid=(grid,),
        in_specs=in_specs,
        out_specs=pl.BlockSpec((BR, F), lambda i: (i, 0)),
        out_shape=jax.ShapeDtypeStruct((_NPAD, F), jnp.float32),
    )


_comb1_128 = _make_comb(_FIN, False)
_comb2_128 = _make_comb(_FIN, True)


_BRM = 2048


def _m1_body(t0, t1, t2, t3, w_ref, v_ref, b_ref, out_ref):
    ts = (t0, t1, t2, t3)
    acc = jnp.zeros((_BRM, _HID), jnp.float32)
    for k in range(_K):
        u = _COS[k] * w_ref[k] + _SIN[k] * v_ref[k]
        acc = acc + jnp.dot(ts[k][:, :], u,
                            preferred_element_type=jnp.float32)
    h = jnp.maximum(acc + b_ref[0, :][None, :], 0.0)
    out_ref[:, :] = jnp.concatenate(
        [h, jnp.zeros((_BRM, _FIN - _HID), jnp.float32)], axis=1)


_m1_call = pl.pallas_call(
    _m1_body,
    grid=(_NPAD // _BRM,),
    in_specs=[
        pl.BlockSpec((_BRM, _FIN), lambda i: (i, 0)),
        pl.BlockSpec((_BRM, _FIN), lambda i: (i, 0)),
        pl.BlockSpec((_BRM, _FIN), lambda i: (i, 0)),
        pl.BlockSpec((_BRM, _FIN), lambda i: (i, 0)),
        pl.BlockSpec((_K, _FIN, _HID), lambda i: (0, 0, 0)),
        pl.BlockSpec((_K, _FIN, _HID), lambda i: (0, 0, 0)),
        pl.BlockSpec((1, _HID), lambda i: (0, 0)),
    ],
    out_specs=pl.BlockSpec((_BRM, _FIN), lambda i: (i, 0)),
    out_shape=jax.ShapeDtypeStruct((_NPAD, _FIN), jnp.float32),
)


def _m2_body(s0, s1, s2, s3, w_ref, v_ref, b_ref, out_ref):
    ss = (s0, s1, s2, s3)
    acc = jnp.zeros((_BRM, _CLS), jnp.float32)
    for k in range(_K):
        u = _COS[k] * w_ref[k] + _SIN[k] * v_ref[k]
        acc = acc + jnp.dot(ss[k][:, :_HID], u,
                            preferred_element_type=jnp.float32)
    lg = acc + b_ref[0, :][None, :]
    m = jnp.max(lg, axis=1, keepdims=True)
    e = lg - m
    out_ref[:, :] = e - jnp.log(jnp.sum(jnp.exp(e), axis=1, keepdims=True))


_m2_call = pl.pallas_call(
    _m2_body,
    grid=(_NPAD // _BRM,),
    in_specs=[
        pl.BlockSpec((_BRM, _FIN), lambda i: (i, 0)),
        pl.BlockSpec((_BRM, _FIN), lambda i: (i, 0)),
        pl.BlockSpec((_BRM, _FIN), lambda i: (i, 0)),
        pl.BlockSpec((_BRM, _FIN), lambda i: (i, 0)),
        pl.BlockSpec((_K, _HID, _CLS), lambda i: (0, 0, 0)),
        pl.BlockSpec((_K, _HID, _CLS), lambda i: (0, 0, 0)),
        pl.BlockSpec((1, _CLS), lambda i: (0, 0)),
    ],
    out_specs=pl.BlockSpec((_BRM, _CLS), lambda i: (i, 0)),
    out_shape=jax.ShapeDtypeStruct((_NPAD, _CLS), jnp.float32),
)


def _pad_edges(a):
    return jnp.pad(a.reshape(_NW, _EREAL),
                   ((0, 0), (0, _EPT - _EREAL))).reshape(-1)


def kernel(edge_index, edge_weight, x, W1, V1, b1, W2, V2, b2):
    src = _pad_edges(edge_index[0])
    dst = _pad_edges(edge_index[1])
    ew = _pad_edges(edge_weight)
    xp = jnp.pad(x, ((0, _NPAD - _N), (0, 0)))

    deg2 = _deg_call(dst, ew)
    dinv = _dinv_call(deg2).reshape(_NPAD)
    packi, packn = _norm_call(src, dst, ew, dinv)

    p = _prop128(packi, packn, xp)
    t1 = _comb1_128(p)
    p = _prop128(packi, packn, t1)
    t2 = _comb2_128(p, xp)
    p = _prop128(packi, packn, t2)
    t3 = _comb2_128(p, t1)
    h = _m1_call(xp, t1, t2, t3, W1, V1, b1.reshape(1, _HID))

    q = _prop128(packi, packn, h)
    s1 = _comb1_128(q)
    q = _prop128(packi, packn, s1)
    s2 = _comb2_128(q, h)
    q = _prop128(packi, packn, s2)
    s3 = _comb2_128(q, s1)
    out = _m2_call(h, s1, s2, s3, W2, V2, b2.reshape(1, _CLS))
    return out[:_N]


# speedup vs baseline: 1.6803x; 1.6803x over previous
import functools
import math

import jax
import jax.numpy as jnp
from jax import lax
from jax.experimental import pallas as pl
from jax.experimental.pallas import tpu as pltpu
from jax.experimental.pallas import tpu_sc as plsc

_N = 10000
_NPAD = 10240
_E = 640000
_FIN = 128
_HID = 64
_CLS = 7
_K = 4

_NC, _NS, _LANES = 2, 16, 16
_NW = _NC * _NS
_B = 80
_EPT = _E // _NW
_NCHUNK = _EPT // _B
_RPT = _NPAD // _NS

_COS = [math.cos(k * math.pi / _K) for k in range(_K)]
_SIN = [math.sin(k * math.pi / _K) for k in range(_K)]


def _mesh():
    return plsc.VectorSubcoreMesh(
        core_axis_name="c", subcore_axis_name="s",
        num_cores=_NC, num_subcores=_NS)


def _deg_body(dst_hbm, w_hbm, out_hbm, dstb, wb, zb, acc, lsem0, lsem1):
    c = lax.axis_index("c")
    s = lax.axis_index("s")
    wid = c * _NS + s
    lsem = (lsem0, lsem1)

    def zloop(i, carry):
        zb[pl.ds(i * 16, 16)] = jnp.zeros((16,), jnp.float32)
        return carry
    lax.fori_loop(0, _RPT // 16, zloop, 0)
    pltpu.sync_copy(zb, acc.at[pl.ds(s * _RPT, _RPT)])
    plsc.subcore_barrier()

    base = wid * _EPT

    def start_loads(i, b):
        off = base + i * _B
        pltpu.async_copy(dst_hbm.at[pl.ds(off, _B)], dstb.at[b], lsem[b])
        pltpu.async_copy(w_hbm.at[pl.ds(off, _B)], wb.at[b], lsem[b])

    def wait_loads(i, b):
        off = base + i * _B
        pltpu.make_async_copy(
            dst_hbm.at[pl.ds(off, _B)], dstb.at[b], lsem[b]).wait()
        pltpu.make_async_copy(
            w_hbm.at[pl.ds(off, _B)], wb.at[b], lsem[b]).wait()

    start_loads(0, 0)
    start_loads(1, 1)

    def macro(m, carry):
        for b in (0, 1):
            i = m * 2 + b
            wait_loads(i, b)
            pltpu.sync_copy(wb.at[b], acc.at[dstb.at[b]], add=True)

            @pl.when(i < _NCHUNK - 2)
            def _():
                start_loads(i + 2, b)
        return carry
    lax.fori_loop(0, _NCHUNK // 2, macro, 0)
    plsc.subcore_barrier()
    pltpu.sync_copy(acc.at[pl.ds(s * _RPT, _RPT)],
                    out_hbm.at[c, pl.ds(s * _RPT, _RPT)])


_deg_call = functools.partial(
    pl.kernel,
    out_type=jax.ShapeDtypeStruct((_NC, _NPAD), jnp.float32),
    mesh=_mesh(),
    scratch_types=[
        pltpu.VMEM((2, _B), jnp.int32),
        pltpu.VMEM((2, _B), jnp.float32),
        pltpu.VMEM((_RPT,), jnp.float32),
        pltpu.VMEM_SHARED((_NPAD,), jnp.float32),
        pltpu.SemaphoreType.DMA,
        pltpu.SemaphoreType.DMA,
    ],
)(_deg_body)


_NCHT = _NW * _NCHUNK


def _norm_body(src_hbm, dst_hbm, w_hbm, dinv_hbm, packi_hbm, packn_hbm,
               srcb, dstb, wb, gsb, gdb, pkb, nb,
               lsem0, lsem1, gsem0, gsem1, osem0, osem1):
    c = lax.axis_index("c")
    s = lax.axis_index("s")
    wid = c * _NS + s
    base = wid * _EPT
    lsem = (lsem0, lsem1)
    gsem = (gsem0, gsem1)
    osem = (osem0, osem1)

    def start_loads(i, b):
        off = base + i * _B
        pltpu.async_copy(src_hbm.at[pl.ds(off, _B)], srcb.at[b], lsem[b])
        pltpu.async_copy(dst_hbm.at[pl.ds(off, _B)], dstb.at[b], lsem[b])
        pltpu.async_copy(w_hbm.at[pl.ds(off, _B)], wb.at[b], lsem[b])

    def wait_loads(i, b):
        off = base + i * _B
        pltpu.make_async_copy(
            src_hbm.at[pl.ds(off, _B)], srcb.at[b], lsem[b]).wait()
        pltpu.make_async_copy(
            dst_hbm.at[pl.ds(off, _B)], dstb.at[b], lsem[b]).wait()
        pltpu.make_async_copy(
            w_hbm.at[pl.ds(off, _B)], wb.at[b], lsem[b]).wait()

    def start_gathers(b):
        pltpu.async_copy(dinv_hbm.at[srcb.at[b]], gsb.at[b], gsem[b])
        pltpu.async_copy(dinv_hbm.at[dstb.at[b]], gdb.at[b], gsem[b])

    def wait_gathers(b):
        pltpu.make_async_copy(
            dinv_hbm.at[srcb.at[b]], gsb.at[b], gsem[b]).wait()
        pltpu.make_async_copy(
            dinv_hbm.at[dstb.at[b]], gdb.at[b], gsem[b]).wait()

    def start_out(i, b):
        cid = wid * _NCHUNK + i
        pltpu.async_copy(pkb.at[b], packi_hbm.at[cid], osem[b])
        pltpu.async_copy(nb.at[b], packn_hbm.at[cid], osem[b])

    def wait_out(i, b):
        cid = wid * _NCHUNK + i
        pltpu.make_async_copy(
            pkb.at[b], packi_hbm.at[cid], osem[b]).wait()
        pltpu.make_async_copy(
            nb.at[b], packn_hbm.at[cid], osem[b]).wait()

    start_loads(0, 0)
    start_loads(1, 1)
    wait_loads(0, 0)
    start_gathers(0)

    def macro(m, carry):
        for b in (0, 1):
            i = m * 2 + b
            q = 1 - b

            @pl.when(i < _NCHUNK - 1)
            def _():
                wait_loads(i + 1, q)
                start_gathers(q)

            wait_gathers(b)

            @pl.when(i >= 2)
            def _():
                wait_out(i - 2, b)

            def vec(k, carry2):
                sl = pl.ds(k * 16, 16)
                pkb[b, 0, sl] = srcb[b, sl]
                pkb[b, 1, sl] = dstb[b, sl]
                nb[b, sl] = gsb[b, sl] * wb[b, sl] * gdb[b, sl]
                return carry2
            lax.fori_loop(0, _B // 16, vec, 0)
            start_out(i, b)

            @pl.when(i < _NCHUNK - 2)
            def _():
                start_loads(i + 2, b)
        return carry
    lax.fori_loop(0, _NCHUNK // 2, macro, 0)
    wait_out(_NCHUNK - 2, 0)
    wait_out(_NCHUNK - 1, 1)


_norm_call = functools.partial(
    pl.kernel,
    out_type=[jax.ShapeDtypeStruct((_NCHT, 2, _B), jnp.int32),
              jax.ShapeDtypeStruct((_NCHT, _B), jnp.float32)],
    mesh=_mesh(),
    scratch_types=[
        pltpu.VMEM((2, _B), jnp.int32),
        pltpu.VMEM((2, _B), jnp.int32),
        pltpu.VMEM((2, _B), jnp.float32),
        pltpu.VMEM((2, _B), jnp.float32),
        pltpu.VMEM((2, _B), jnp.float32),
        pltpu.VMEM((2, 2, _B), jnp.int32),
        pltpu.VMEM((2, _B), jnp.float32),
        pltpu.SemaphoreType.DMA,
        pltpu.SemaphoreType.DMA,
        pltpu.SemaphoreType.DMA,
        pltpu.SemaphoreType.DMA,
        pltpu.SemaphoreType.DMA,
        pltpu.SemaphoreType.DMA,
    ],
)(_norm_body)


def _make_prop(F):
    def body(packi_hbm, packn_hbm, h_hbm, out_hbm,
             pk, pn, rows, dstc, acc, *sems):
        c = lax.axis_index("c")
        s = lax.axis_index("s")
        wid = c * _NS + s
        psem = sems[0:2]
        gsem = sems[2:4]
        ssem = sems[4:6]

        def zr_loop(i, carry):
            for j in range(F // 16):
                rows[0, i, pl.ds(j * 16, 16)] = jnp.zeros((16,), jnp.float32)
            return carry
        lax.fori_loop(0, _B, zr_loop, 0)

        def zcp(kk, carry):
            pltpu.sync_copy(rows.at[0],
                            acc.at[pl.ds(s * _RPT + kk * _B, _B)])
            return carry
        lax.fori_loop(0, _RPT // _B, zcp, 0)
        plsc.subcore_barrier()

        cbase = wid * _NCHUNK

        def start_pack(i, b):
            pltpu.async_copy(packi_hbm.at[cbase + i], pk.at[b], psem[b])
            pltpu.async_copy(packn_hbm.at[cbase + i], pn.at[b], psem[b])

        def wait_pack(i, b):
            pltpu.make_async_copy(
                packi_hbm.at[cbase + i], pk.at[b], psem[b]).wait()
            pltpu.make_async_copy(
                packn_hbm.at[cbase + i], pn.at[b], psem[b]).wait()

        def start_gather(b):
            pltpu.async_copy(h_hbm.at[pk.at[b, 0]], rows.at[b], gsem[b])

        def wait_gather(b):
            pltpu.make_async_copy(
                h_hbm.at[pk.at[b, 0]], rows.at[b], gsem[b]).wait()

        def start_scatter(b):
            pltpu.async_copy(rows.at[b], acc.at[dstc.at[b]], ssem[b],
                             add=True)

        def wait_scatter(b):
            pltpu.make_async_copy(rows.at[b], acc.at[dstc.at[b]],
                                  ssem[b]).wait()

        start_pack(0, 0)
        start_pack(1, 1)
        wait_pack(0, 0)
        start_gather(0)

        def macro(m, carry):
            for b in (0, 1):
                i = m * 2 + b
                q = 1 - b

                @pl.when(i > 0)
                def _():
                    wait_scatter(q)

                @pl.when(i < _NCHUNK - 1)
                def _():
                    wait_pack(i + 1, q)
                    start_gather(q)

                wait_gather(b)

                def scale(g, carry2):
                    sl = pl.ds(g * 16, 16)
                    dstc[b, sl] = pk[b, 1, sl]
                    nv16 = pn[b, sl]
                    for r in range(16):
                        nv = nv16[r]
                        row = g * 16 + r
                        for j in range(F // 16):
                            slj = pl.ds(j * 16, 16)
                            rows[b, row, slj] = rows[b, row, slj] * nv
                    return carry2
                lax.fori_loop(0, _B // 16, scale, 0)

                start_scatter(b)

                @pl.when(i < _NCHUNK - 2)
                def _():
                    start_pack(i + 2, b)
            return carry
        lax.fori_loop(0, _NCHUNK // 2, macro, 0)
        wait_scatter((_NCHUNK - 1) % 2)
        plsc.subcore_barrier()
        pltpu.sync_copy(acc.at[pl.ds(s * _RPT, _RPT)],
                        out_hbm.at[c, pl.ds(s * _RPT, _RPT)])

    return functools.partial(
        pl.kernel,
        out_type=jax.ShapeDtypeStruct((_NC, _NPAD, F), jnp.float32),
        mesh=_mesh(),
        scratch_types=[
            pltpu.VMEM((2, 2, _B), jnp.int32),
            pltpu.VMEM((2, _B), jnp.float32),
            pltpu.VMEM((2, _B, F), jnp.float32),
            pltpu.VMEM((2, _B), jnp.int32),
            pltpu.VMEM_SHARED((_NPAD, F), jnp.float32),
        ] + [pltpu.SemaphoreType.DMA] * 6,
    )(body)


_prop128 = _make_prop(_FIN)


def _dinv_body(deg_ref, out_ref):
    d = deg_ref[0:1, :] + deg_ref[1:2, :]
    out_ref[:, :] = jnp.where(
        d > 0, lax.rsqrt(jnp.maximum(d, 1e-12)), 0.0)


_dinv_call = pl.pallas_call(
    _dinv_body,
    out_shape=jax.ShapeDtypeStruct((1, _NPAD), jnp.float32),
)


def _make_comb(F, with_base):
    BR = 2048
    grid = _NPAD // BR

    if with_base:
        def body(p_ref, base_ref, out_ref):
            ssum = p_ref[0, :, :] + p_ref[1, :, :]
            out_ref[:, :] = -2.0 * ssum - base_ref[:, :]
        in_specs = [
            pl.BlockSpec((2, BR, F), lambda i: (0, i, 0)),
            pl.BlockSpec((BR, F), lambda i: (i, 0)),
        ]
    else:
        def body(p_ref, out_ref):
            out_ref[:, :] = -(p_ref[0, :, :] + p_ref[1, :, :])
        in_specs = [pl.BlockSpec((2, BR, F), lambda i: (0, i, 0))]

    return pl.pallas_call(
        body,
        grid=(grid,),
        in_specs=in_specs,
        out_specs=pl.BlockSpec((BR, F), lambda i: (i, 0)),
        out_shape=jax.ShapeDtypeStruct((_NPAD, F), jnp.float32),
    )


_comb1_128 = _make_comb(_FIN, False)
_comb2_128 = _make_comb(_FIN, True)


_BRM = 2048


def _m1_body(t0, t1, t2, t3, w_ref, v_ref, b_ref, out_ref):
    ts = (t0, t1, t2, t3)
    acc = jnp.zeros((_BRM, _HID), jnp.float32)
    for k in range(_K):
        u = _COS[k] * w_ref[k] + _SIN[k] * v_ref[k]
        acc = acc + jnp.dot(ts[k][:, :], u,
                            preferred_element_type=jnp.float32)
    h = jnp.maximum(acc + b_ref[0, :][None, :], 0.0)
    out_ref[:, :] = jnp.concatenate(
        [h, jnp.zeros((_BRM, _FIN - _HID), jnp.float32)], axis=1)


_m1_call = pl.pallas_call(
    _m1_body,
    grid=(_NPAD // _BRM,),
    in_specs=[
        pl.BlockSpec((_BRM, _FIN), lambda i: (i, 0)),
        pl.BlockSpec((_BRM, _FIN), lambda i: (i, 0)),
        pl.BlockSpec((_BRM, _FIN), lambda i: (i, 0)),
        pl.BlockSpec((_BRM, _FIN), lambda i: (i, 0)),
        pl.BlockSpec((_K, _FIN, _HID), lambda i: (0, 0, 0)),
        pl.BlockSpec((_K, _FIN, _HID), lambda i: (0, 0, 0)),
        pl.BlockSpec((1, _HID), lambda i: (0, 0)),
    ],
    out_specs=pl.BlockSpec((_BRM, _FIN), lambda i: (i, 0)),
    out_shape=jax.ShapeDtypeStruct((_NPAD, _FIN), jnp.float32),
)


def _m2_body(s0, s1, s2, s3, w_ref, v_ref, b_ref, out_ref):
    ss = (s0, s1, s2, s3)
    acc = jnp.zeros((_BRM, _CLS), jnp.float32)
    for k in range(_K):
        u = _COS[k] * w_ref[k] + _SIN[k] * v_ref[k]
        acc = acc + jnp.dot(ss[k][:, :_HID], u,
                            preferred_element_type=jnp.float32)
    lg = acc + b_ref[0, :][None, :]
    m = jnp.max(lg, axis=1, keepdims=True)
    e = lg - m
    out_ref[:, :] = e - jnp.log(jnp.sum(jnp.exp(e), axis=1, keepdims=True))


_m2_call = pl.pallas_call(
    _m2_body,
    grid=(_NPAD // _BRM,),
    in_specs=[
        pl.BlockSpec((_BRM, _FIN), lambda i: (i, 0)),
        pl.BlockSpec((_BRM, _FIN), lambda i: (i, 0)),
        pl.BlockSpec((_BRM, _FIN), lambda i: (i, 0)),
        pl.BlockSpec((_BRM, _FIN), lambda i: (i, 0)),
        pl.BlockSpec((_K, _HID, _CLS), lambda i: (0, 0, 0)),
        pl.BlockSpec((_K, _HID, _CLS), lambda i: (0, 0, 0)),
        pl.BlockSpec((1, _CLS), lambda i: (0, 0)),
    ],
    out_specs=pl.BlockSpec((_BRM, _CLS), lambda i: (i, 0)),
    out_shape=jax.ShapeDtypeStruct((_NPAD, _CLS), jnp.float32),
)


def kernel(edge_index, edge_weight, x, W1, V1, b1, W2, V2, b2):
    src = edge_index[0]
    dst = edge_index[1]
    ew = edge_weight
    xp = jnp.pad(x, ((0, _NPAD - _N), (0, 0)))

    deg2 = _deg_call(dst, ew)
    dinv = _dinv_call(deg2).reshape(_NPAD)
    packi, packn = _norm_call(src, dst, ew, dinv)

    p = _prop128(packi, packn, xp)
    t1 = _comb1_128(p)
    p = _prop128(packi, packn, t1)
    t2 = _comb2_128(p, xp)
    p = _prop128(packi, packn, t2)
    t3 = _comb2_128(p, t1)
    h = _m1_call(xp, t1, t2, t3, W1, V1, b1.reshape(1, _HID))

    q = _prop128(packi, packn, h)
    s1 = _comb1_128(q)
    q = _prop128(packi, packn, s1)
    s2 = _comb2_128(q, h)
    q = _prop128(packi, packn, s2)
    s3 = _comb2_128(q, s1)
    out = _m2_call(h, s1, s2, s3, W2, V2, b2.reshape(1, _CLS))
    return out[:_N]


# speedup vs baseline: 1.7111x; 1.0183x over previous
import functools
import math

import jax
import jax.numpy as jnp
from jax import lax
from jax.experimental import pallas as pl
from jax.experimental.pallas import tpu as pltpu
from jax.experimental.pallas import tpu_sc as plsc

_N = 10000
_NPAD = 10240
_E = 640000
_FIN = 128
_HID = 64
_CLS = 7
_K = 4

_NC, _NS, _LANES = 2, 16, 16
_NW = _NC * _NS
_B = 80
_EPT = _E // _NW
_NCHUNK = _EPT // _B
_RPT = _NPAD // _NS

_COS = [math.cos(k * math.pi / _K) for k in range(_K)]
_SIN = [math.sin(k * math.pi / _K) for k in range(_K)]


def _mesh():
    return plsc.VectorSubcoreMesh(
        core_axis_name="c", subcore_axis_name="s",
        num_cores=_NC, num_subcores=_NS)


_DR = 5


def _deg_body(dst_hbm, w_hbm, out_hbm, dstb, wb, zb, acc, *sems):
    c = lax.axis_index("c")
    s = lax.axis_index("s")
    wid = c * _NS + s
    lsem = sems[0:_DR]
    ssem = sems[_DR:2 * _DR]

    def zloop(i, carry):
        zb[pl.ds(i * 16, 16)] = jnp.zeros((16,), jnp.float32)
        return carry
    lax.fori_loop(0, _RPT // 16, zloop, 0)
    pltpu.sync_copy(zb, acc.at[pl.ds(s * _RPT, _RPT)])
    plsc.subcore_barrier()

    base = wid * _EPT

    def start_loads(i, b):
        off = base + i * _B
        pltpu.async_copy(dst_hbm.at[pl.ds(off, _B)], dstb.at[b], lsem[b])
        pltpu.async_copy(w_hbm.at[pl.ds(off, _B)], wb.at[b], lsem[b])

    def wait_loads(i, b):
        off = base + i * _B
        pltpu.make_async_copy(
            dst_hbm.at[pl.ds(off, _B)], dstb.at[b], lsem[b]).wait()
        pltpu.make_async_copy(
            w_hbm.at[pl.ds(off, _B)], wb.at[b], lsem[b]).wait()

    def start_scatter(b):
        pltpu.async_copy(wb.at[b], acc.at[dstb.at[b]], ssem[b], add=True)

    def wait_scatter(b):
        pltpu.make_async_copy(wb.at[b], acc.at[dstb.at[b]], ssem[b]).wait()

    for b in range(_DR - 1):
        start_loads(b, b)

    def macro(m, carry):
        for b in range(_DR):
            i = m * _DR + b
            wait_loads(i, b)
            start_scatter(b)

            @pl.when(i > 0)
            def _():
                wait_scatter((b + _DR - 1) % _DR)

            @pl.when(i + _DR - 1 < _NCHUNK)
            def _():
                start_loads(i + _DR - 1, (b + _DR - 1) % _DR)
        return carry
    lax.fori_loop(0, _NCHUNK // _DR, macro, 0)
    wait_scatter((_NCHUNK - 1) % _DR)
    plsc.subcore_barrier()
    pltpu.sync_copy(acc.at[pl.ds(s * _RPT, _RPT)],
                    out_hbm.at[c, pl.ds(s * _RPT, _RPT)])


_deg_call = functools.partial(
    pl.kernel,
    out_type=jax.ShapeDtypeStruct((_NC, _NPAD), jnp.float32),
    mesh=_mesh(),
    scratch_types=[
        pltpu.VMEM((_DR, _B), jnp.int32),
        pltpu.VMEM((_DR, _B), jnp.float32),
        pltpu.VMEM((_RPT,), jnp.float32),
        pltpu.VMEM_SHARED((_NPAD,), jnp.float32),
    ] + [pltpu.SemaphoreType.DMA] * (2 * _DR),
)(_deg_body)


_NCHT = _NW * _NCHUNK


def _norm_body(src_hbm, dst_hbm, w_hbm, dinv_hbm, packi_hbm, packn_hbm,
               srcb, dstb, wb, gsb, gdb, pkb, nb,
               lsem0, lsem1, gsem0, gsem1, osem0, osem1):
    c = lax.axis_index("c")
    s = lax.axis_index("s")
    wid = c * _NS + s
    base = wid * _EPT
    lsem = (lsem0, lsem1)
    gsem = (gsem0, gsem1)
    osem = (osem0, osem1)

    def start_loads(i, b):
        off = base + i * _B
        pltpu.async_copy(src_hbm.at[pl.ds(off, _B)], srcb.at[b], lsem[b])
        pltpu.async_copy(dst_hbm.at[pl.ds(off, _B)], dstb.at[b], lsem[b])
        pltpu.async_copy(w_hbm.at[pl.ds(off, _B)], wb.at[b], lsem[b])

    def wait_loads(i, b):
        off = base + i * _B
        pltpu.make_async_copy(
            src_hbm.at[pl.ds(off, _B)], srcb.at[b], lsem[b]).wait()
        pltpu.make_async_copy(
            dst_hbm.at[pl.ds(off, _B)], dstb.at[b], lsem[b]).wait()
        pltpu.make_async_copy(
            w_hbm.at[pl.ds(off, _B)], wb.at[b], lsem[b]).wait()

    def start_gathers(b):
        pltpu.async_copy(dinv_hbm.at[srcb.at[b]], gsb.at[b], gsem[b])
        pltpu.async_copy(dinv_hbm.at[dstb.at[b]], gdb.at[b], gsem[b])

    def wait_gathers(b):
        pltpu.make_async_copy(
            dinv_hbm.at[srcb.at[b]], gsb.at[b], gsem[b]).wait()
        pltpu.make_async_copy(
            dinv_hbm.at[dstb.at[b]], gdb.at[b], gsem[b]).wait()

    def start_out(i, b):
        cid = wid * _NCHUNK + i
        pltpu.async_copy(pkb.at[b], packi_hbm.at[cid], osem[b])
        pltpu.async_copy(nb.at[b], packn_hbm.at[cid], osem[b])

    def wait_out(i, b):
        cid = wid * _NCHUNK + i
        pltpu.make_async_copy(
            pkb.at[b], packi_hbm.at[cid], osem[b]).wait()
        pltpu.make_async_copy(
            nb.at[b], packn_hbm.at[cid], osem[b]).wait()

    start_loads(0, 0)
    start_loads(1, 1)
    wait_loads(0, 0)
    start_gathers(0)

    def macro(m, carry):
        for b in (0, 1):
            i = m * 2 + b
            q = 1 - b

            @pl.when(i < _NCHUNK - 1)
            def _():
                wait_loads(i + 1, q)
                start_gathers(q)

            wait_gathers(b)

            @pl.when(i >= 2)
            def _():
                wait_out(i - 2, b)

            def vec(k, carry2):
                sl = pl.ds(k * 16, 16)
                pkb[b, 0, sl] = srcb[b, sl]
                pkb[b, 1, sl] = dstb[b, sl]
                nb[b, sl] = gsb[b, sl] * wb[b, sl] * gdb[b, sl]
                return carry2
            lax.fori_loop(0, _B // 16, vec, 0)
            start_out(i, b)

            @pl.when(i < _NCHUNK - 2)
            def _():
                start_loads(i + 2, b)
        return carry
    lax.fori_loop(0, _NCHUNK // 2, macro, 0)
    wait_out(_NCHUNK - 2, 0)
    wait_out(_NCHUNK - 1, 1)


_norm_call = functools.partial(
    pl.kernel,
    out_type=[jax.ShapeDtypeStruct((_NCHT, 2, _B), jnp.int32),
              jax.ShapeDtypeStruct((_NCHT, _B), jnp.float32)],
    mesh=_mesh(),
    scratch_types=[
        pltpu.VMEM((2, _B), jnp.int32),
        pltpu.VMEM((2, _B), jnp.int32),
        pltpu.VMEM((2, _B), jnp.float32),
        pltpu.VMEM((2, _B), jnp.float32),
        pltpu.VMEM((2, _B), jnp.float32),
        pltpu.VMEM((2, 2, _B), jnp.int32),
        pltpu.VMEM((2, _B), jnp.float32),
        pltpu.SemaphoreType.DMA,
        pltpu.SemaphoreType.DMA,
        pltpu.SemaphoreType.DMA,
        pltpu.SemaphoreType.DMA,
        pltpu.SemaphoreType.DMA,
        pltpu.SemaphoreType.DMA,
    ],
)(_norm_body)


def _make_prop(F):
    def body(packi_hbm, packn_hbm, h_hbm, out_hbm,
             pk, pn, rows, dstc, acc, *sems):
        c = lax.axis_index("c")
        s = lax.axis_index("s")
        wid = c * _NS + s
        psem = sems[0:2]
        gsem = sems[2:4]
        ssem = sems[4:6]

        def zr_loop(i, carry):
            for j in range(F // 16):
                rows[0, i, pl.ds(j * 16, 16)] = jnp.zeros((16,), jnp.float32)
            return carry
        lax.fori_loop(0, _B, zr_loop, 0)

        def zcp(kk, carry):
            pltpu.sync_copy(rows.at[0],
                            acc.at[pl.ds(s * _RPT + kk * _B, _B)])
            return carry
        lax.fori_loop(0, _RPT // _B, zcp, 0)
        plsc.subcore_barrier()

        cbase = wid * _NCHUNK

        def start_pack(i, b):
            pltpu.async_copy(packi_hbm.at[cbase + i], pk.at[b], psem[b])
            pltpu.async_copy(packn_hbm.at[cbase + i], pn.at[b], psem[b])

        def wait_pack(i, b):
            pltpu.make_async_copy(
                packi_hbm.at[cbase + i], pk.at[b], psem[b]).wait()
            pltpu.make_async_copy(
                packn_hbm.at[cbase + i], pn.at[b], psem[b]).wait()

        def start_gather(b):
            pltpu.async_copy(h_hbm.at[pk.at[b, 0]], rows.at[b], gsem[b])

        def wait_gather(b):
            pltpu.make_async_copy(
                h_hbm.at[pk.at[b, 0]], rows.at[b], gsem[b]).wait()

        def start_scatter(b):
            pltpu.async_copy(rows.at[b], acc.at[dstc.at[b]], ssem[b],
                             add=True)

        def wait_scatter(b):
            pltpu.make_async_copy(rows.at[b], acc.at[dstc.at[b]],
                                  ssem[b]).wait()

        start_pack(0, 0)
        start_pack(1, 1)
        wait_pack(0, 0)
        start_gather(0)

        def macro(m, carry):
            for b in (0, 1):
                i = m * 2 + b
                q = 1 - b

                @pl.when(i > 0)
                def _():
                    wait_scatter(q)

                @pl.when(i < _NCHUNK - 1)
                def _():
                    wait_pack(i + 1, q)
                    start_gather(q)

                wait_gather(b)

                def scale(g, carry2):
                    sl = pl.ds(g * 16, 16)
                    dstc[b, sl] = pk[b, 1, sl]
                    nv16 = pn[b, sl]
                    for r in range(16):
                        nv = nv16[r]
                        row = g * 16 + r
                        for j in range(F // 16):
                            slj = pl.ds(j * 16, 16)
                            rows[b, row, slj] = rows[b, row, slj] * nv
                    return carry2
                lax.fori_loop(0, _B // 16, scale, 0)

                start_scatter(b)

                @pl.when(i < _NCHUNK - 2)
                def _():
                    start_pack(i + 2, b)
            return carry
        lax.fori_loop(0, _NCHUNK // 2, macro, 0)
        wait_scatter((_NCHUNK - 1) % 2)
        plsc.subcore_barrier()
        pltpu.sync_copy(acc.at[pl.ds(s * _RPT, _RPT)],
                        out_hbm.at[c, pl.ds(s * _RPT, _RPT)])

    return functools.partial(
        pl.kernel,
        out_type=jax.ShapeDtypeStruct((_NC, _NPAD, F), jnp.float32),
        mesh=_mesh(),
        scratch_types=[
            pltpu.VMEM((2, 2, _B), jnp.int32),
            pltpu.VMEM((2, _B), jnp.float32),
            pltpu.VMEM((2, _B, F), jnp.float32),
            pltpu.VMEM((2, _B), jnp.int32),
            pltpu.VMEM_SHARED((_NPAD, F), jnp.float32),
        ] + [pltpu.SemaphoreType.DMA] * 6,
    )(body)


_prop128 = _make_prop(_FIN)


def _dinv_body(deg_ref, out_ref):
    d = deg_ref[0:1, :] + deg_ref[1:2, :]
    out_ref[:, :] = jnp.where(
        d > 0, lax.rsqrt(jnp.maximum(d, 1e-12)), 0.0)


_dinv_call = pl.pallas_call(
    _dinv_body,
    out_shape=jax.ShapeDtypeStruct((1, _NPAD), jnp.float32),
)


def _make_comb(F, with_base):
    BR = 2048
    grid = _NPAD // BR

    if with_base:
        def body(p_ref, base_ref, out_ref):
            ssum = p_ref[0, :, :] + p_ref[1, :, :]
            out_ref[:, :] = -2.0 * ssum - base_ref[:, :]
        in_specs = [
            pl.BlockSpec((2, BR, F), lambda i: (0, i, 0)),
            pl.BlockSpec((BR, F), lambda i: (i, 0)),
        ]
    else:
        def body(p_ref, out_ref):
            out_ref[:, :] = -(p_ref[0, :, :] + p_ref[1, :, :])
        in_specs = [pl.BlockSpec((2, BR, F), lambda i: (0, i, 0))]

    return pl.pallas_call(
        body,
        grid=(grid,),
        in_specs=in_specs,
        out_specs=pl.BlockSpec((BR, F), lambda i: (i, 0)),
        out_shape=jax.ShapeDtypeStruct((_NPAD, F), jnp.float32),
    )


_comb1_128 = _make_comb(_FIN, False)
_comb2_128 = _make_comb(_FIN, True)


_BRM = 2048


def _m1_body(t0, t1, t2, t3, w_ref, v_ref, b_ref, out_ref):
    ts = (t0, t1, t2, t3)
    acc = jnp.zeros((_BRM, _HID), jnp.float32)
    for k in range(_K):
        u = _COS[k] * w_ref[k] + _SIN[k] * v_ref[k]
        acc = acc + jnp.dot(ts[k][:, :], u,
                            preferred_element_type=jnp.float32)
    h = jnp.maximum(acc + b_ref[0, :][None, :], 0.0)
    out_ref[:, :] = jnp.concatenate(
        [h, jnp.zeros((_BRM, _FIN - _HID), jnp.float32)], axis=1)


_m1_call = pl.pallas_call(
    _m1_body,
    grid=(_NPAD // _BRM,),
    in_specs=[
        pl.BlockSpec((_BRM, _FIN), lambda i: (i, 0)),
        pl.BlockSpec((_BRM, _FIN), lambda i: (i, 0)),
        pl.BlockSpec((_BRM, _FIN), lambda i: (i, 0)),
        pl.BlockSpec((_BRM, _FIN), lambda i: (i, 0)),
        pl.BlockSpec((_K, _FIN, _HID), lambda i: (0, 0, 0)),
        pl.BlockSpec((_K, _FIN, _HID), lambda i: (0, 0, 0)),
        pl.BlockSpec((1, _HID), lambda i: (0, 0)),
    ],
    out_specs=pl.BlockSpec((_BRM, _FIN), lambda i: (i, 0)),
    out_shape=jax.ShapeDtypeStruct((_NPAD, _FIN), jnp.float32),
)


def _m2_body(s0, s1, s2, s3, w_ref, v_ref, b_ref, out_ref):
    ss = (s0, s1, s2, s3)
    acc = jnp.zeros((_BRM, _CLS), jnp.float32)
    for k in range(_K):
        u = _COS[k] * w_ref[k] + _SIN[k] * v_ref[k]
        acc = acc + jnp.dot(ss[k][:, :_HID], u,
                            preferred_element_type=jnp.float32)
    lg = acc + b_ref[0, :][None, :]
    m = jnp.max(lg, axis=1, keepdims=True)
    e = lg - m
    out_ref[:, :] = e - jnp.log(jnp.sum(jnp.exp(e), axis=1, keepdims=True))


_m2_call = pl.pallas_call(
    _m2_body,
    grid=(_NPAD // _BRM,),
    in_specs=[
        pl.BlockSpec((_BRM, _FIN), lambda i: (i, 0)),
        pl.BlockSpec((_BRM, _FIN), lambda i: (i, 0)),
        pl.BlockSpec((_BRM, _FIN), lambda i: (i, 0)),
        pl.BlockSpec((_BRM, _FIN), lambda i: (i, 0)),
        pl.BlockSpec((_K, _HID, _CLS), lambda i: (0, 0, 0)),
        pl.BlockSpec((_K, _HID, _CLS), lambda i: (0, 0, 0)),
        pl.BlockSpec((1, _CLS), lambda i: (0, 0)),
    ],
    out_specs=pl.BlockSpec((_BRM, _CLS), lambda i: (i, 0)),
    out_shape=jax.ShapeDtypeStruct((_NPAD, _CLS), jnp.float32),
)


def kernel(edge_index, edge_weight, x, W1, V1, b1, W2, V2, b2):
    src = edge_index[0]
    dst = edge_index[1]
    ew = edge_weight
    xp = jnp.pad(x, ((0, _NPAD - _N), (0, 0)))

    deg2 = _deg_call(dst, ew)
    dinv = _dinv_call(deg2).reshape(_NPAD)
    packi, packn = _norm_call(src, dst, ew, dinv)

    p = _prop128(packi, packn, xp)
    t1 = _comb1_128(p)
    p = _prop128(packi, packn, t1)
    t2 = _comb2_128(p, xp)
    p = _prop128(packi, packn, t2)
    t3 = _comb2_128(p, t1)
    h = _m1_call(xp, t1, t2, t3, W1, V1, b1.reshape(1, _HID))

    q = _prop128(packi, packn, h)
    s1 = _comb1_128(q)
    q = _prop128(packi, packn, s1)
    s2 = _comb2_128(q, h)
    q = _prop128(packi, packn, s2)
    s3 = _comb2_128(q, s1)
    out = _m2_call(h, s1, s2, s3, W2, V2, b2.reshape(1, _CLS))
    return out[:_N]
